# baseline (speedup 1.0000x reference)
import jax
import jax.numpy as jnp
from jax.experimental import pallas as pl
from jax.experimental.pallas import tpu as pltpu

B, S, D_IN, D_OUT = 4, 2048, 4096, 4096
R = 16
COEF = 0.2 * (32 / 16)
M = B * S

TM = 256
TN = 2048


def _fused_kernel(uid_ref, x_ref, wt_ref, b_ref, acatt_ref, bcatt_ref,
                  uemb_ref, wrut_ref, out_ref, u_scr):
    j = pl.program_id(0)
    i = pl.program_id(1)

    @pl.when(j == 0)
    def _prelude():
        tall = jax.lax.dot_general(
            x_ref[...], acatt_ref[...], (((1,), (0,)), ((), ())),
            preferred_element_type=jnp.float32)
        pin1 = jax.nn.sigmoid(tall[:, 33:34] - tall[:, 32:33])
        u2 = uemb_ref[...].reshape(1, D_IN)
        zu = jax.lax.dot_general(
            u2, wrut_ref[...], (((1,), (0,)), ((), ())),
            preferred_element_type=jnp.float32)
        pu1 = jax.nn.sigmoid(zu[:, 1:2] - zu[:, 0:1])
        g1 = jax.nn.sigmoid(2.0 * (pin1 + pu1) - 2.0)
        u0 = (COEF * (1.0 - g1)) * tall[:, 0:16]
        u1 = (COEF * g1) * tall[:, 16:32]
        u_scr[pl.ds(i * TM, TM), :] = jnp.concatenate([u0, u1], axis=1)

    acc = jax.lax.dot_general(
        x_ref[...], wt_ref[...], (((1,), (0,)), ((), ())),
        preferred_element_type=jnp.float32)
    uv = u_scr[pl.ds(i * TM, TM), :]
    lora = jax.lax.dot_general(
        uv, bcatt_ref[...], (((1,), (0,)), ((), ())),
        preferred_element_type=jnp.float32)
    out_ref[...] = acc + b_ref[...] + lora


def kernel(x, user_id, W, b, W_route_in, W_route_user, user_emb,
           A0, B0, A1, B1):
    xb = x.reshape(M, D_IN).astype(jnp.bfloat16)
    Wt = W.T.astype(jnp.bfloat16)
    acatt = jnp.concatenate(
        [A0, A1, W_route_in, jnp.zeros((6, D_IN), jnp.float32)],
        axis=0).T.astype(jnp.bfloat16)
    bcatt = jnp.concatenate([B0, B1], axis=1).T
    wrut = W_route_user.T
    b2 = b.reshape(1, D_OUT)
    uemb3 = user_emb.reshape(user_emb.shape[0], 1, D_IN)
    uid = (user_id[0] - 1).astype(jnp.int32).reshape(1)

    grid = (D_OUT // TN, M // TM)
    out = pl.pallas_call(
        _fused_kernel,
        grid_spec=pltpu.PrefetchScalarGridSpec(
            num_scalar_prefetch=1,
            grid=grid,
            in_specs=[
                pl.BlockSpec((TM, D_IN), lambda j, i, u: (i, 0)),
                pl.BlockSpec((D_IN, TN), lambda j, i, u: (0, j)),
                pl.BlockSpec((1, TN), lambda j, i, u: (0, j)),
                pl.BlockSpec((D_IN, 40), lambda j, i, u: (0, 0)),
                pl.BlockSpec((32, TN), lambda j, i, u: (0, j)),
                pl.BlockSpec((1, 1, D_IN), lambda j, i, u: (u[0], 0, 0)),
                pl.BlockSpec((D_IN, 2), lambda j, i, u: (0, 0)),
            ],
            out_specs=pl.BlockSpec((TM, TN), lambda j, i, u: (i, j)),
            scratch_shapes=[pltpu.VMEM((M, 32), jnp.float32)],
        ),
        out_shape=jax.ShapeDtypeStruct((M, D_OUT), jnp.float32),
        compiler_params=pltpu.CompilerParams(
            dimension_semantics=("arbitrary", "arbitrary"),
        ),
    )(uid, xb, Wt, b2, acatt, bcatt, uemb3, wrut)
    return out.reshape(B, S, D_OUT)


# speedup vs baseline: 1.0237x; 1.0237x over previous
import jax
import jax.numpy as jnp
from jax.experimental import pallas as pl
from jax.experimental.pallas import tpu as pltpu

B, S, D_IN, D_OUT = 4, 2048, 4096, 4096
R = 16
COEF = 0.2 * (32 / 16)
M = B * S

TM = 512
TN = 1024


def _fused_kernel(uid_ref, x_ref, wt_ref, b_ref, acatt_ref, bcatt_ref,
                  uemb_ref, wrut_ref, out_ref, u_scr):
    j = pl.program_id(0)
    i = pl.program_id(1)

    @pl.when(j == 0)
    def _prelude():
        tall = jax.lax.dot_general(
            x_ref[...], acatt_ref[...], (((1,), (0,)), ((), ())),
            preferred_element_type=jnp.float32)
        pin1 = jax.nn.sigmoid(tall[:, 33:34] - tall[:, 32:33])
        u2 = uemb_ref[...].reshape(1, D_IN)
        zu = jax.lax.dot_general(
            u2, wrut_ref[...], (((1,), (0,)), ((), ())),
            preferred_element_type=jnp.float32)
        pu1 = jax.nn.sigmoid(zu[:, 1:2] - zu[:, 0:1])
        g1 = jax.nn.sigmoid(2.0 * (pin1 + pu1) - 2.0)
        u0 = (COEF * (1.0 - g1)) * tall[:, 0:16]
        u1 = (COEF * g1) * tall[:, 16:32]
        u_scr[pl.ds(i * TM, TM), :] = jnp.concatenate([u0, u1], axis=1)

    acc = jax.lax.dot_general(
        x_ref[...], wt_ref[...], (((1,), (0,)), ((), ())),
        preferred_element_type=jnp.float32)
    uv = u_scr[pl.ds(i * TM, TM), :]
    lora = jax.lax.dot_general(
        uv, bcatt_ref[...], (((1,), (0,)), ((), ())),
        preferred_element_type=jnp.float32)
    out_ref[...] = acc + b_ref[...] + lora


def kernel(x, user_id, W, b, W_route_in, W_route_user, user_emb,
           A0, B0, A1, B1):
    xb = x.reshape(M, D_IN).astype(jnp.bfloat16)
    Wt = W.T.astype(jnp.bfloat16)
    acatt = jnp.concatenate(
        [A0, A1, W_route_in, jnp.zeros((6, D_IN), jnp.float32)],
        axis=0).T.astype(jnp.bfloat16)
    bcatt = jnp.concatenate([B0, B1], axis=1).T
    wrut = W_route_user.T
    b2 = b.reshape(1, D_OUT)
    uemb3 = user_emb.reshape(user_emb.shape[0], 1, D_IN)
    uid = (user_id[0] - 1).astype(jnp.int32).reshape(1)

    grid = (D_OUT // TN, M // TM)
    out = pl.pallas_call(
        _fused_kernel,
        grid_spec=pltpu.PrefetchScalarGridSpec(
            num_scalar_prefetch=1,
            grid=grid,
            in_specs=[
                pl.BlockSpec((TM, D_IN), lambda j, i, u: (i, 0)),
                pl.BlockSpec((D_IN, TN), lambda j, i, u: (0, j)),
                pl.BlockSpec((1, TN), lambda j, i, u: (0, j)),
                pl.BlockSpec((D_IN, 40), lambda j, i, u: (0, 0)),
                pl.BlockSpec((32, TN), lambda j, i, u: (0, j)),
                pl.BlockSpec((1, 1, D_IN), lambda j, i, u: (u[0], 0, 0)),
                pl.BlockSpec((D_IN, 2), lambda j, i, u: (0, 0)),
            ],
            out_specs=pl.BlockSpec((TM, TN), lambda j, i, u: (i, j)),
            scratch_shapes=[pltpu.VMEM((M, 32), jnp.float32)],
        ),
        out_shape=jax.ShapeDtypeStruct((M, D_OUT), jnp.float32),
        compiler_params=pltpu.CompilerParams(
            dimension_semantics=("arbitrary", "arbitrary"),
        ),
    )(uid, xb, Wt, b2, acatt, bcatt, uemb3, wrut)
    return out.reshape(B, S, D_OUT)


# speedup vs baseline: 1.1639x; 1.1369x over previous
import jax
import jax.numpy as jnp
from jax.experimental import pallas as pl
from jax.experimental.pallas import tpu as pltpu

B, S, D_IN, D_OUT = 4, 2048, 4096, 4096
R = 16
COEF = 0.2 * (32 / 16)
M = B * S

TM = 256


def _fused_kernel(uid_ref, x_ref, w_ref, b_ref, acat_ref, bcat_ref,
                  uemb_ref, wru_ref, out_ref):
    xc = x_ref[...].astype(jnp.bfloat16)
    tall = jax.lax.dot_general(
        xc, acat_ref[...], (((1,), (1,)), ((), ())),
        preferred_element_type=jnp.float32)
    pin1 = jax.nn.sigmoid(tall[:, 33:34] - tall[:, 32:33])
    u2 = uemb_ref[...].reshape(1, D_IN)
    zu = jax.lax.dot_general(
        u2, wru_ref[...], (((1,), (1,)), ((), ())),
        preferred_element_type=jnp.float32)
    pu1 = jax.nn.sigmoid(zu[:, 1:2] - zu[:, 0:1])
    g1 = jax.nn.sigmoid(2.0 * (pin1 + pu1) - 2.0)
    u0 = (COEF * (1.0 - g1)) * tall[:, 0:16]
    u1 = (COEF * g1) * tall[:, 16:32]
    uv = jnp.concatenate([u0, u1], axis=1)

    acc = jax.lax.dot_general(
        xc, w_ref[...], (((1,), (1,)), ((), ())),
        preferred_element_type=jnp.float32)
    lora = jax.lax.dot_general(
        uv, bcat_ref[...], (((1,), (1,)), ((), ())),
        preferred_element_type=jnp.float32)
    out_ref[...] = acc + b_ref[...] + lora


def kernel(x, user_id, W, b, W_route_in, W_route_user, user_emb,
           A0, B0, A1, B1):
    x2 = x.reshape(M, D_IN)
    Wb = W.astype(jnp.bfloat16)
    acat = jnp.concatenate(
        [A0, A1, W_route_in, jnp.zeros((6, D_IN), jnp.float32)],
        axis=0).astype(jnp.bfloat16)
    bcat = jnp.concatenate([B0, B1], axis=1)
    b2 = b.reshape(1, D_OUT)
    uemb3 = user_emb.reshape(user_emb.shape[0], 1, D_IN)
    uid = (user_id[0] - 1).astype(jnp.int32).reshape(1)

    grid = (M // TM,)
    out = pl.pallas_call(
        _fused_kernel,
        grid_spec=pltpu.PrefetchScalarGridSpec(
            num_scalar_prefetch=1,
            grid=grid,
            in_specs=[
                pl.BlockSpec((TM, D_IN), lambda i, u: (i, 0)),
                pl.BlockSpec((D_OUT, D_IN), lambda i, u: (0, 0)),
                pl.BlockSpec((1, D_OUT), lambda i, u: (0, 0)),
                pl.BlockSpec((40, D_IN), lambda i, u: (0, 0)),
                pl.BlockSpec((D_OUT, 32), lambda i, u: (0, 0)),
                pl.BlockSpec((1, 1, D_IN), lambda i, u: (u[0], 0, 0)),
                pl.BlockSpec((2, D_IN), lambda i, u: (0, 0)),
            ],
            out_specs=pl.BlockSpec((TM, D_OUT), lambda i, u: (i, 0)),
            scratch_shapes=[],
        ),
        out_shape=jax.ShapeDtypeStruct((M, D_OUT), jnp.float32),
        compiler_params=pltpu.CompilerParams(
            dimension_semantics=("arbitrary",),
        ),
    )(uid, x2, Wb, b2, acat, bcat, uemb3, W_route_user)
    return out.reshape(B, S, D_OUT)
